# baseline (speedup 1.0000x reference)
import jax
import jax.numpy as jnp
from jax.experimental import pallas as pl
from jax.experimental.pallas import tpu as pltpu

_GRID = 64


def _copy_body(k_ref, v_ref, ok_ref, ov_ref):
    ok_ref[...] = k_ref[...]
    ov_ref[...] = v_ref[...]


def kernel(new_k, new_v):
    B, H, L, HD = new_k.shape
    rows = B * H * L // _GRID
    k2 = new_k.reshape(_GRID, rows, HD)
    v2 = new_v.reshape(_GRID, rows, HD)
    out_shape = (
        jax.ShapeDtypeStruct(k2.shape, k2.dtype),
        jax.ShapeDtypeStruct(v2.shape, v2.dtype),
    )
    spec = pl.BlockSpec((1, rows, HD), lambda i: (i, 0, 0))
    ok, ov = pl.pallas_call(
        _copy_body,
        grid=(_GRID,),
        out_shape=out_shape,
        in_specs=[spec, spec],
        out_specs=[spec, spec],
        compiler_params=pltpu.CompilerParams(
            dimension_semantics=("parallel",),
        ),
    )(k2, v2)
    return ok.reshape(B, H, L, HD), ov.reshape(B, H, L, HD)


# speedup vs baseline: 1.0161x; 1.0161x over previous
import jax
import jax.numpy as jnp
from jax.experimental import pallas as pl
from jax.experimental.pallas import tpu as pltpu

_GRID = 32


def _copy_body(k_ref, v_ref, ok_ref, ov_ref):
    ok_ref[...] = k_ref[...]
    ov_ref[...] = v_ref[...]


def kernel(new_k, new_v):
    B, H, L, HD = new_k.shape
    rows = B * H * L // _GRID
    k2 = new_k.reshape(_GRID, rows, HD)
    v2 = new_v.reshape(_GRID, rows, HD)
    out_shape = (
        jax.ShapeDtypeStruct(k2.shape, k2.dtype),
        jax.ShapeDtypeStruct(v2.shape, v2.dtype),
    )
    spec = pl.BlockSpec((1, rows, HD), lambda i: (i, 0, 0))
    ok, ov = pl.pallas_call(
        _copy_body,
        grid=(_GRID,),
        out_shape=out_shape,
        in_specs=[spec, spec],
        out_specs=[spec, spec],
        compiler_params=pltpu.CompilerParams(
            dimension_semantics=("parallel",),
        ),
    )(k2, v2)
    return ok.reshape(B, H, L, HD), ov.reshape(B, H, L, HD)
